# initial kernel scaffold (unmeasured)
import jax
import jax.numpy as jnp
from jax import lax
from jax.experimental import pallas as pl
from jax.experimental.pallas import tpu as pltpu

N_DEV = 4
T = 8

_CompilerParams = getattr(pltpu, "CompilerParams", None) or getattr(
    pltpu, "TPUCompilerParams"
)


def _gelu(y):
    c = 0.7978845608028654
    return 0.5 * y * (1.0 + jnp.tanh(c * (y + 0.044715 * y * y * y)))


def kernel(x, w_mat):
    M, Ks = x.shape
    _, N = w_mat.shape
    MT = M // N_DEV
    nt = N // T

    def body(x_ref, w_ref, out_ref, send_bufs, recv_bufs,
             send_sems, recv_sems, credit_sem):
        t = pl.program_id(0)
        p = lax.axis_index("i")
        right = lax.rem(p + 1, N_DEV)
        left = lax.rem(p + N_DEV - 1, N_DEV)

        @pl.when(t == 0)
        def _():
            bsem = pltpu.get_barrier_semaphore()
            for nbr in (left, right):
                pltpu.semaphore_signal(
                    bsem, inc=1, device_id=(nbr,),
                    device_id_type=pltpu.DeviceIdType.MESH,
                )
            pltpu.semaphore_wait(bsem, 2)

        def partial_chunk(c):
            xa = x_ref[pl.ds(c * MT, MT), :]
            return jnp.dot(xa, w_ref[:, :], preferred_element_type=jnp.float32)

        rdmas = [
            pltpu.make_async_remote_copy(
                src_ref=send_bufs.at[g],
                dst_ref=recv_bufs.at[g],
                send_sem=send_sems.at[g],
                recv_sem=recv_sems.at[g],
                device_id=(right,),
                device_id_type=pltpu.DeviceIdType.MESH,
            )
            for g in range(N_DEV - 1)
        ]

        def signal_credit_left():
            pltpu.semaphore_signal(
                credit_sem, inc=1, device_id=(left,),
                device_id_type=pltpu.DeviceIdType.MESH,
            )

        @pl.when(t > 0)
        def _():
            rdmas[0].wait_send()
        send_bufs[0, :, :] = partial_chunk(lax.rem(p + 3, N_DEV))

        @pl.when(t > 0)
        def _():
            pltpu.semaphore_wait(credit_sem, 1)
        rdmas[0].start()

        @pl.when(t > 0)
        def _():
            rdmas[1].wait_send()
        send_bufs[1, :, :] = partial_chunk(lax.rem(p + 2, N_DEV))
        rdmas[0].wait_recv()
        send_bufs[1, :, :] = send_bufs[1, :, :] + recv_bufs[0, :, :]

        @pl.when(t > 0)
        def _():
            pltpu.semaphore_wait(credit_sem, 1)
        rdmas[1].start()
        signal_credit_left()

        @pl.when(t > 0)
        def _():
            rdmas[2].wait_send()
        send_bufs[2, :, :] = partial_chunk(lax.rem(p + 1, N_DEV))
        rdmas[1].wait_recv()
        send_bufs[2, :, :] = send_bufs[2, :, :] + recv_bufs[1, :, :]

        @pl.when(t > 0)
        def _():
            pltpu.semaphore_wait(credit_sem, 1)
        rdmas[2].start()
        signal_credit_left()

        out_ref[:, :] = partial_chunk(p)
        rdmas[2].wait_recv()
        out_ref[:, :] = _gelu(out_ref[:, :] + recv_bufs[2, :, :])
        signal_credit_left()

        @pl.when(t == T - 1)
        def _():
            for g in range(N_DEV - 1):
                rdmas[g].wait_send()
            pltpu.semaphore_wait(credit_sem, N_DEV - 1)

    return pl.pallas_call(
        body,
        grid=(T,),
        out_shape=jax.ShapeDtypeStruct((MT, N), jnp.float32),
        in_specs=[
            pl.BlockSpec(memory_space=pltpu.VMEM),
            pl.BlockSpec((Ks, nt), lambda t: (0, t)),
        ],
        out_specs=pl.BlockSpec((MT, nt), lambda t: (0, t)),
        scratch_shapes=[
            pltpu.VMEM((N_DEV - 1, MT, nt), jnp.float32),
            pltpu.VMEM((N_DEV - 1, MT, nt), jnp.float32),
            pltpu.SemaphoreType.DMA((N_DEV - 1,)),
            pltpu.SemaphoreType.DMA((N_DEV - 1,)),
            pltpu.SemaphoreType.REGULAR,
        ],
        compiler_params=_CompilerParams(
            collective_id=0,
            dimension_semantics=("arbitrary",),
        ),
    )(x, w_mat)


# baseline (device time: 1194864 ns/iter reference)
import jax
import jax.numpy as jnp
from jax import lax
from jax.experimental import pallas as pl
from jax.experimental.pallas import tpu as pltpu

N_DEV = 4
T = 8

_CompilerParams = getattr(pltpu, "CompilerParams", None) or getattr(
    pltpu, "TPUCompilerParams"
)


def _gelu(y):
    c = 0.7978845608028654
    return 0.5 * y * (1.0 + jnp.tanh(c * (y + 0.044715 * y * y * y)))


def kernel(x, w_mat):
    M, Ks = x.shape
    _, N = w_mat.shape
    MT = M // N_DEV
    nt = N // T

    def body(x_ref, w_ref, out_ref, send_bufs, recv_bufs,
             send_sems, recv_sems, credit_sem):
        t = pl.program_id(0)
        p = lax.axis_index("i")
        right = lax.rem(p + 1, N_DEV)
        left = lax.rem(p + N_DEV - 1, N_DEV)

        @pl.when(t == 0)
        def _():
            bsem = pltpu.get_barrier_semaphore()
            for nbr in (left, right):
                pltpu.semaphore_signal(
                    bsem, inc=1, device_id=(nbr,),
                    device_id_type=pltpu.DeviceIdType.MESH,
                )
            pltpu.semaphore_wait(bsem, 2)

        def partial_chunk(c):
            xa = x_ref[pl.ds(c * MT, MT), :]
            return jnp.dot(xa, w_ref[:, :], preferred_element_type=jnp.float32)

        rdmas = [
            pltpu.make_async_remote_copy(
                src_ref=send_bufs.at[g],
                dst_ref=recv_bufs.at[g],
                send_sem=send_sems.at[g],
                recv_sem=recv_sems.at[g],
                device_id=(right,),
                device_id_type=pltpu.DeviceIdType.MESH,
            )
            for g in range(N_DEV - 1)
        ]

        def signal_credit_left():
            pltpu.semaphore_signal(
                credit_sem, inc=1, device_id=(left,),
                device_id_type=pltpu.DeviceIdType.MESH,
            )

        @pl.when(t > 0)
        def _():
            rdmas[0].wait_send()
        send_bufs[0, :, :] = partial_chunk(lax.rem(p + 3, N_DEV))

        @pl.when(t > 0)
        def _():
            pltpu.semaphore_wait(credit_sem, 1)
        rdmas[0].start()

        @pl.when(t > 0)
        def _():
            rdmas[1].wait_send()
        send_bufs[1, :, :] = partial_chunk(lax.rem(p + 2, N_DEV))
        rdmas[0].wait_recv()
        send_bufs[1, :, :] = send_bufs[1, :, :] + recv_bufs[0, :, :]

        @pl.when(t > 0)
        def _():
            pltpu.semaphore_wait(credit_sem, 1)
        rdmas[1].start()
        signal_credit_left()

        @pl.when(t > 0)
        def _():
            rdmas[2].wait_send()
        send_bufs[2, :, :] = partial_chunk(lax.rem(p + 1, N_DEV))
        rdmas[1].wait_recv()
        send_bufs[2, :, :] = send_bufs[2, :, :] + recv_bufs[1, :, :]

        @pl.when(t > 0)
        def _():
            pltpu.semaphore_wait(credit_sem, 1)
        rdmas[2].start()
        signal_credit_left()

        out_ref[:, :] = partial_chunk(p)
        rdmas[2].wait_recv()
        out_ref[:, :] = _gelu(out_ref[:, :] + recv_bufs[2, :, :])
        signal_credit_left()

        @pl.when(t == T - 1)
        def _():
            for g in range(N_DEV - 1):
                rdmas[g].wait_send()
            pltpu.semaphore_wait(credit_sem, N_DEV - 1)

    return pl.pallas_call(
        body,
        grid=(T,),
        out_shape=jax.ShapeDtypeStruct((MT, N), jnp.float32),
        in_specs=[
            pl.BlockSpec(memory_space=pltpu.VMEM),
            pl.BlockSpec((Ks, nt), lambda t: (0, t)),
        ],
        out_specs=pl.BlockSpec((MT, nt), lambda t: (0, t)),
        scratch_shapes=[
            pltpu.VMEM((N_DEV - 1, MT, nt), jnp.float32),
            pltpu.VMEM((N_DEV - 1, MT, nt), jnp.float32),
            pltpu.SemaphoreType.DMA((N_DEV - 1,)),
            pltpu.SemaphoreType.DMA((N_DEV - 1,)),
            pltpu.SemaphoreType.REGULAR,
        ],
        compiler_params=_CompilerParams(
            collective_id=0,
            dimension_semantics=("arbitrary",),
            vmem_limit_bytes=64 * 1024 * 1024,
        ),
    )(x, w_mat)


# device time: 653637 ns/iter; 1.8280x vs baseline; 1.8280x over previous
import jax
import jax.numpy as jnp
from jax import lax
from jax.experimental import pallas as pl
from jax.experimental.pallas import tpu as pltpu

N_DEV = 4
T = 8

_CompilerParams = getattr(pltpu, "CompilerParams", None) or getattr(
    pltpu, "TPUCompilerParams"
)


def _gelu(y):
    c = 0.7978845608028654
    return 0.5 * y * (1.0 + jnp.tanh(c * (y + 0.044715 * y * y * y)))


def kernel(x, w_mat):
    M, Ks = x.shape
    _, N = w_mat.shape
    MT = M // N_DEV
    nt = N // T
    nh = nt // 2

    def body(x_ref, w_ref, out_ref,
             send_a, recv_a, send_b, recv_b,
             ss_a, rs_a, ss_b, rs_b, cred_a, cred_b):
        t = pl.program_id(0)
        p = lax.axis_index("i")
        right = lax.rem(p + 1, N_DEV)
        left = lax.rem(p + N_DEV - 1, N_DEV)

        @pl.when(t == 0)
        def _():
            bsem = pltpu.get_barrier_semaphore()
            for nbr in (left, right):
                pltpu.semaphore_signal(
                    bsem, inc=1, device_id=(nbr,),
                    device_id_type=pltpu.DeviceIdType.MESH,
                )
            pltpu.semaphore_wait(bsem, 2)

        def partial_chunk(c, col0):
            xa = x_ref[pl.ds(c * MT, MT), :]
            return jnp.dot(xa, w_ref[:, pl.ds(col0, nh)],
                           preferred_element_type=jnp.float32)

        class Dir:
            pass

        dir_a = Dir()
        dir_a.send_buf, dir_a.recv_buf = send_a, recv_a
        dir_a.to, dir_a.frm = right, left
        dir_a.cred = cred_a
        dir_a.chunk = lambda g: lax.rem(p + N_DEV - 1 - g + N_DEV, N_DEV)
        dir_a.col0 = 0
        dir_a.ss, dir_a.rs = ss_a, rs_a

        dir_b = Dir()
        dir_b.send_buf, dir_b.recv_buf = send_b, recv_b
        dir_b.to, dir_b.frm = left, right
        dir_b.cred = cred_b
        dir_b.chunk = lambda g: lax.rem(p + 1 + g, N_DEV)
        dir_b.col0 = nh
        dir_b.ss, dir_b.rs = ss_b, rs_b

        dirs = (dir_a, dir_b)
        for d in dirs:
            d.rdmas = [
                pltpu.make_async_remote_copy(
                    src_ref=d.send_buf.at[g],
                    dst_ref=d.recv_buf.at[g],
                    send_sem=d.ss.at[g],
                    recv_sem=d.rs.at[g],
                    device_id=(d.to,),
                    device_id_type=pltpu.DeviceIdType.MESH,
                )
                for g in range(N_DEV - 1)
            ]

        def signal_credit(d):
            pltpu.semaphore_signal(
                d.cred, inc=1, device_id=(d.frm,),
                device_id_type=pltpu.DeviceIdType.MESH,
            )

        for g in range(N_DEV - 1):
            for d in dirs:
                @pl.when(t > 0)
                def _(d=d, g=g):
                    d.rdmas[g].wait_send()
                d.send_buf[g, :, :] = partial_chunk(d.chunk(g), d.col0)
            for d in dirs:
                if g > 0:
                    d.rdmas[g - 1].wait_recv()
                    d.send_buf[g, :, :] = (
                        d.send_buf[g, :, :] + d.recv_buf[g - 1, :, :]
                    )

                @pl.when(t > 0)
                def _(d=d):
                    pltpu.semaphore_wait(d.cred, 1)
                d.rdmas[g].start()
                if g > 0:
                    signal_credit(d)

        for d in dirs:
            out_ref[:, pl.ds(d.col0, nh)] = partial_chunk(p, d.col0)
        for d in dirs:
            d.rdmas[N_DEV - 2].wait_recv()
            out_ref[:, pl.ds(d.col0, nh)] = _gelu(
                out_ref[:, pl.ds(d.col0, nh)] + d.recv_buf[N_DEV - 2, :, :]
            )
            signal_credit(d)

        @pl.when(t == T - 1)
        def _():
            for d in dirs:
                for g in range(N_DEV - 1):
                    d.rdmas[g].wait_send()
                pltpu.semaphore_wait(d.cred, N_DEV - 1)

    return pl.pallas_call(
        body,
        grid=(T,),
        out_shape=jax.ShapeDtypeStruct((MT, N), jnp.float32),
        in_specs=[
            pl.BlockSpec(memory_space=pltpu.VMEM),
            pl.BlockSpec((Ks, nt), lambda t: (0, t)),
        ],
        out_specs=pl.BlockSpec((MT, nt), lambda t: (0, t)),
        scratch_shapes=[
            pltpu.VMEM((N_DEV - 1, MT, nh), jnp.float32),
            pltpu.VMEM((N_DEV - 1, MT, nh), jnp.float32),
            pltpu.VMEM((N_DEV - 1, MT, nh), jnp.float32),
            pltpu.VMEM((N_DEV - 1, MT, nh), jnp.float32),
            pltpu.SemaphoreType.DMA((N_DEV - 1,)),
            pltpu.SemaphoreType.DMA((N_DEV - 1,)),
            pltpu.SemaphoreType.DMA((N_DEV - 1,)),
            pltpu.SemaphoreType.DMA((N_DEV - 1,)),
            pltpu.SemaphoreType.REGULAR,
            pltpu.SemaphoreType.REGULAR,
        ],
        compiler_params=_CompilerParams(
            collective_id=0,
            dimension_semantics=("arbitrary",),
            vmem_limit_bytes=64 * 1024 * 1024,
        ),
    )(x, w_mat)


# device time: 611195 ns/iter; 1.9550x vs baseline; 1.0694x over previous
import jax
import jax.numpy as jnp
from jax import lax
from jax.experimental import pallas as pl
from jax.experimental.pallas import tpu as pltpu

N_DEV = 4
T = 8

_CompilerParams = getattr(pltpu, "CompilerParams", None) or getattr(
    pltpu, "TPUCompilerParams"
)


def _gelu(y):
    c = 0.7978845608028654
    return 0.5 * y * (1.0 + jnp.tanh(c * (y + 0.044715 * y * y * y)))


def kernel(x, w_mat):
    M, Ks = x.shape
    _, N = w_mat.shape
    MT = M // N_DEV
    nt = N // T
    nh = nt // 2

    def body(x_ref, w_ref, out_ref,
             send_a, recv_a, send_b, recv_b, own_buf,
             ss_a, rs_a, ss_b, rs_b, cred_a, cred_b):
        t = pl.program_id(0)
        p = lax.axis_index("i")
        right = lax.rem(p + 1, N_DEV)
        left = lax.rem(p + N_DEV - 1, N_DEV)
        active = t < T
        steady = (t > 0) & (t < T)

        @pl.when(t == 0)
        def _():
            bsem = pltpu.get_barrier_semaphore()
            for nbr in (left, right):
                pltpu.semaphore_signal(
                    bsem, inc=1, device_id=(nbr,),
                    device_id_type=pltpu.DeviceIdType.MESH,
                )
            pltpu.semaphore_wait(bsem, 2)

        def partial_chunk(c, col0):
            xa = x_ref[pl.ds(c * MT, MT), :]
            return jnp.dot(xa, w_ref[:, pl.ds(col0, nh)],
                           preferred_element_type=jnp.float32)

        class Dir:
            pass

        dir_a = Dir()
        dir_a.send_buf, dir_a.recv_buf = send_a, recv_a
        dir_a.to, dir_a.frm = right, left
        dir_a.cred = cred_a
        dir_a.chunk = lambda g: lax.rem(p + N_DEV - 1 - g + N_DEV, N_DEV)
        dir_a.col0 = 0
        dir_a.ss, dir_a.rs = ss_a, rs_a

        dir_b = Dir()
        dir_b.send_buf, dir_b.recv_buf = send_b, recv_b
        dir_b.to, dir_b.frm = left, right
        dir_b.cred = cred_b
        dir_b.chunk = lambda g: lax.rem(p + 1 + g, N_DEV)
        dir_b.col0 = nh
        dir_b.ss, dir_b.rs = ss_b, rs_b

        dirs = (dir_a, dir_b)
        for d in dirs:
            d.rdmas = [
                pltpu.make_async_remote_copy(
                    src_ref=d.send_buf.at[g],
                    dst_ref=d.recv_buf.at[g],
                    send_sem=d.ss.at[g],
                    recv_sem=d.rs.at[g],
                    device_id=(d.to,),
                    device_id_type=pltpu.DeviceIdType.MESH,
                )
                for g in range(N_DEV - 1)
            ]

        def signal_credit(d):
            pltpu.semaphore_signal(
                d.cred, inc=1, device_id=(d.frm,),
                device_id_type=pltpu.DeviceIdType.MESH,
            )

        for d in dirs:
            @pl.when(steady)
            def _(d=d):
                d.rdmas[0].wait_send()

            @pl.when(active)
            def _(d=d):
                d.send_buf[0, :, :] = partial_chunk(d.chunk(0), d.col0)

            @pl.when(steady)
            def _(d=d):
                pltpu.semaphore_wait(d.cred, 1)

            @pl.when(active)
            def _(d=d):
                d.rdmas[0].start()

        @pl.when(t > 0)
        def _():
            for d in dirs:
                d.rdmas[N_DEV - 2].wait_recv()
                out_ref[:, pl.ds(d.col0, nh)] = _gelu(
                    own_buf[:, pl.ds(d.col0, nh)]
                    + d.recv_buf[N_DEV - 2, :, :]
                )
                signal_credit(d)

        for g in range(1, N_DEV - 1):
            for d in dirs:
                @pl.when(steady)
                def _(d=d, g=g):
                    d.rdmas[g].wait_send()

                @pl.when(active)
                def _(d=d, g=g):
                    d.send_buf[g, :, :] = partial_chunk(d.chunk(g), d.col0)
            for d in dirs:
                @pl.when(active)
                def _(d=d, g=g):
                    d.rdmas[g - 1].wait_recv()
                    d.send_buf[g, :, :] = (
                        d.send_buf[g, :, :] + d.recv_buf[g - 1, :, :]
                    )

                @pl.when(steady)
                def _(d=d):
                    pltpu.semaphore_wait(d.cred, 1)

                @pl.when(active)
                def _(d=d, g=g):
                    d.rdmas[g].start()
                    signal_credit(d)

        @pl.when(active)
        def _():
            xa = x_ref[pl.ds(p * MT, MT), :]
            own_buf[:, :] = jnp.dot(xa, w_ref[:, :],
                                    preferred_element_type=jnp.float32)

        @pl.when(t == T)
        def _():
            for d in dirs:
                for g in range(N_DEV - 1):
                    d.rdmas[g].wait_send()
                pltpu.semaphore_wait(d.cred, N_DEV - 1)

    return pl.pallas_call(
        body,
        grid=(T + 1,),
        out_shape=jax.ShapeDtypeStruct((MT, N), jnp.float32),
        in_specs=[
            pl.BlockSpec(memory_space=pltpu.VMEM),
            pl.BlockSpec((Ks, nt),
                         lambda t: (0, jnp.minimum(t, T - 1))),
        ],
        out_specs=pl.BlockSpec((MT, nt),
                               lambda t: (0, jnp.maximum(t - 1, 0))),
        scratch_shapes=[
            pltpu.VMEM((N_DEV - 1, MT, nh), jnp.float32),
            pltpu.VMEM((N_DEV - 1, MT, nh), jnp.float32),
            pltpu.VMEM((N_DEV - 1, MT, nh), jnp.float32),
            pltpu.VMEM((N_DEV - 1, MT, nh), jnp.float32),
            pltpu.VMEM((MT, nt), jnp.float32),
            pltpu.SemaphoreType.DMA((N_DEV - 1,)),
            pltpu.SemaphoreType.DMA((N_DEV - 1,)),
            pltpu.SemaphoreType.DMA((N_DEV - 1,)),
            pltpu.SemaphoreType.DMA((N_DEV - 1,)),
            pltpu.SemaphoreType.REGULAR,
            pltpu.SemaphoreType.REGULAR,
        ],
        compiler_params=_CompilerParams(
            collective_id=0,
            dimension_semantics=("arbitrary",),
            vmem_limit_bytes=64 * 1024 * 1024,
        ),
    )(x, w_mat)


# device time: 580111 ns/iter; 2.0597x vs baseline; 1.0536x over previous
import jax
import jax.numpy as jnp
from jax import lax
from jax.experimental import pallas as pl
from jax.experimental.pallas import tpu as pltpu

N_DEV = 4
T = 8
NSUB = 2

_CompilerParams = getattr(pltpu, "CompilerParams", None) or getattr(
    pltpu, "TPUCompilerParams"
)


def _gelu(y):
    c = 0.7978845608028654
    return 0.5 * y * (1.0 + jnp.tanh(c * (y + 0.044715 * y * y * y)))


def kernel(x, w_mat):
    M, Ks = x.shape
    _, N = w_mat.shape
    MT = M // N_DEV
    nt = N // T
    nh = nt // 2
    hh = MT // NSUB

    def body(x_ref, w_ref, out_ref,
             send_a, recv_a, send_b, recv_b, own_buf,
             ss_a, rs_a, ss_b, rs_b, cred_a, cred_b):
        t = pl.program_id(0)
        p = lax.axis_index("i")
        right = lax.rem(p + 1, N_DEV)
        left = lax.rem(p + N_DEV - 1, N_DEV)
        active = t < T
        steady = (t > 0) & (t < T)

        @pl.when(t == 0)
        def _():
            bsem = pltpu.get_barrier_semaphore()
            for nbr in (left, right):
                pltpu.semaphore_signal(
                    bsem, inc=1, device_id=(nbr,),
                    device_id_type=pltpu.DeviceIdType.MESH,
                )
            pltpu.semaphore_wait(bsem, 2)

        def partial_chunk(c, col0):
            xa = x_ref[pl.ds(c * MT, MT), :]
            return jnp.dot(xa, w_ref[:, pl.ds(col0, nh)],
                           preferred_element_type=jnp.float32)

        class Dir:
            pass

        dir_a = Dir()
        dir_a.send_buf, dir_a.recv_buf = send_a, recv_a
        dir_a.to, dir_a.frm = right, left
        dir_a.cred = cred_a
        dir_a.chunk = lambda g: lax.rem(p + N_DEV - 1 - g + N_DEV, N_DEV)
        dir_a.col0 = 0
        dir_a.ss, dir_a.rs = ss_a, rs_a

        dir_b = Dir()
        dir_b.send_buf, dir_b.recv_buf = send_b, recv_b
        dir_b.to, dir_b.frm = left, right
        dir_b.cred = cred_b
        dir_b.chunk = lambda g: lax.rem(p + 1 + g, N_DEV)
        dir_b.col0 = nh
        dir_b.ss, dir_b.rs = ss_b, rs_b

        dirs = (dir_a, dir_b)
        for d in dirs:
            d.rdmas = [
                [
                    pltpu.make_async_remote_copy(
                        src_ref=d.send_buf.at[g, pl.ds(k * hh, hh), :],
                        dst_ref=d.recv_buf.at[g, pl.ds(k * hh, hh), :],
                        send_sem=d.ss.at[g, k],
                        recv_sem=d.rs.at[g, k],
                        device_id=(d.to,),
                        device_id_type=pltpu.DeviceIdType.MESH,
                    )
                    for k in range(NSUB)
                ]
                for g in range(N_DEV - 1)
            ]

        def signal_credit(d):
            pltpu.semaphore_signal(
                d.cred, inc=1, device_id=(d.frm,),
                device_id_type=pltpu.DeviceIdType.MESH,
            )

        for d in dirs:
            @pl.when(steady)
            def _(d=d):
                for k in range(NSUB):
                    d.rdmas[0][k].wait_send()

            @pl.when(active)
            def _(d=d):
                d.send_buf[0, :, :] = partial_chunk(d.chunk(0), d.col0)

            @pl.when(steady)
            def _(d=d):
                pltpu.semaphore_wait(d.cred, 1)

            @pl.when(active)
            def _(d=d):
                for k in range(NSUB):
                    d.rdmas[0][k].start()

        @pl.when(t > 0)
        def _():
            for k in range(NSUB):
                for d in dirs:
                    d.rdmas[N_DEV - 2][k].wait_recv()
                    rows = pl.ds(k * hh, hh)
                    out_ref[rows, pl.ds(d.col0, nh)] = _gelu(
                        own_buf[rows, pl.ds(d.col0, nh)]
                        + d.recv_buf[N_DEV - 2, rows, :]
                    )
            for d in dirs:
                signal_credit(d)

        for g in range(1, N_DEV - 1):
            for d in dirs:
                @pl.when(steady)
                def _(d=d, g=g):
                    for k in range(NSUB):
                        d.rdmas[g][k].wait_send()

                @pl.when(active)
                def _(d=d, g=g):
                    d.send_buf[g, :, :] = partial_chunk(d.chunk(g), d.col0)

                @pl.when(steady)
                def _(d=d):
                    pltpu.semaphore_wait(d.cred, 1)
            for k in range(NSUB):
                for d in dirs:
                    @pl.when(active)
                    def _(d=d, g=g, k=k):
                        d.rdmas[g - 1][k].wait_recv()
                        rows = pl.ds(k * hh, hh)
                        d.send_buf[g, rows, :] = (
                            d.send_buf[g, rows, :]
                            + d.recv_buf[g - 1, rows, :]
                        )
                        d.rdmas[g][k].start()
            for d in dirs:
                @pl.when(active)
                def _(d=d):
                    signal_credit(d)

        @pl.when(active)
        def _():
            xa = x_ref[pl.ds(p * MT, MT), :]
            own_buf[:, :] = jnp.dot(xa, w_ref[:, :],
                                    preferred_element_type=jnp.float32)

        @pl.when(t == T)
        def _():
            for d in dirs:
                for g in range(N_DEV - 1):
                    for k in range(NSUB):
                        d.rdmas[g][k].wait_send()
                pltpu.semaphore_wait(d.cred, N_DEV - 1)

    return pl.pallas_call(
        body,
        grid=(T + 1,),
        out_shape=jax.ShapeDtypeStruct((MT, N), jnp.float32),
        in_specs=[
            pl.BlockSpec(memory_space=pltpu.VMEM),
            pl.BlockSpec((Ks, nt),
                         lambda t: (0, jnp.minimum(t, T - 1))),
        ],
        out_specs=pl.BlockSpec((MT, nt),
                               lambda t: (0, jnp.maximum(t - 1, 0))),
        scratch_shapes=[
            pltpu.VMEM((N_DEV - 1, MT, nh), jnp.float32),
            pltpu.VMEM((N_DEV - 1, MT, nh), jnp.float32),
            pltpu.VMEM((N_DEV - 1, MT, nh), jnp.float32),
            pltpu.VMEM((N_DEV - 1, MT, nh), jnp.float32),
            pltpu.VMEM((MT, nt), jnp.float32),
            pltpu.SemaphoreType.DMA((N_DEV - 1, NSUB)),
            pltpu.SemaphoreType.DMA((N_DEV - 1, NSUB)),
            pltpu.SemaphoreType.DMA((N_DEV - 1, NSUB)),
            pltpu.SemaphoreType.DMA((N_DEV - 1, NSUB)),
            pltpu.SemaphoreType.REGULAR,
            pltpu.SemaphoreType.REGULAR,
        ],
        compiler_params=_CompilerParams(
            collective_id=0,
            dimension_semantics=("arbitrary",),
            vmem_limit_bytes=64 * 1024 * 1024,
        ),
    )(x, w_mat)
